# baseline (speedup 1.0000x reference)
import jax
import jax.numpy as jnp
from jax import lax
from jax.experimental import pallas as pl
from jax.experimental.pallas import tpu as pltpu
from jax.experimental.pallas import tpu_sc as plsc

SEQ_LEN = 200
N_DIMS = 64
BATCH = 4096
HALF = SEQ_LEN // 2
HALF_PAD = 104
PAIR = 2 * N_DIMS

NUM_CORES = 2
NUM_SUBCORES = 16
NUM_WORKERS = NUM_CORES * NUM_SUBCORES
ROWS_PER_WORKER = BATCH // NUM_WORKERS

NBUF_G = 4
NBUF_O = 2
LOOKAHEAD = 2


def _body(resp_hbm, pos_hbm, tab_hbm, out_hbm, idx_all, gbuf, obuf, pos_v,
          sem_g, sem_o):
    wid = lax.axis_index("s") * NUM_CORES + lax.axis_index("c")
    b0 = wid * ROWS_PER_WORKER

    pltpu.sync_copy(pos_hbm, pos_v)
    pltpu.sync_copy(resp_hbm.at[pl.ds(2 * b0, 2 * ROWS_PER_WORKER), :], idx_all)

    def out_slice(j):
        b = b0 + j
        return out_hbm.at[b // 8, pl.ds(0, HALF), pl.ds(b % 8, 1), :]

    def gather_descs(j, kg):
        return [
            pltpu.make_async_copy(
                tab_hbm.at[idx_all.at[2 * j + h]],
                gbuf.at[kg, h],
                sem_g.at[kg],
            )
            for h in (0, 1)
        ]

    def start_gathers(j, kg):
        for d in gather_descs(j, kg):
            d.start()

    def wait_gathers(j, kg):
        for d in gather_descs(j, kg):
            d.wait()

    for j in range(LOOKAHEAD):
        start_gathers(j, j)

    def step(i, carry):
        j0 = i * NBUF_G
        for k in range(NBUF_G):
            j = j0 + k
            ko = k % NBUF_O
            wait_gathers(j, k)

            @pl.when(j - NBUF_O >= 0)
            def _():
                pltpu.make_async_copy(
                    obuf.at[ko], out_slice(j - NBUF_O), sem_o.at[ko]
                ).wait()

            @pl.when(j + LOOKAHEAD <= ROWS_PER_WORKER - 1)
            def _():
                start_gathers(j + LOOKAHEAD, (k + LOOKAHEAD) % NBUF_G)

            def add_row(r, c):
                for l in range(PAIR // 16):
                    sl = pl.ds(16 * l, 16)
                    gsl = pl.ds(16 * (l % 4), 16)
                    obuf[ko, r, 0, sl] = (
                        gbuf[k, l // 4, r, gsl] + pos_v[r, sl]
                    )
                return c

            lax.fori_loop(0, HALF, add_row, 0, unroll=4)

            pltpu.async_copy(obuf.at[ko], out_slice(j), sem_o.at[ko])
        return carry

    lax.fori_loop(0, ROWS_PER_WORKER // NBUF_G, step, 0)
    for j in range(ROWS_PER_WORKER - NBUF_O, ROWS_PER_WORKER):
        ko = j % NBUF_O
        pltpu.make_async_copy(obuf.at[ko], out_slice(j), sem_o.at[ko]).wait()


@jax.jit
def _run(resp_prep, pos_pair, response_embed):
    mesh = plsc.VectorSubcoreMesh(core_axis_name="c", subcore_axis_name="s")
    f = pl.kernel(
        _body,
        out_type=jax.ShapeDtypeStruct((BATCH // 8, HALF, 8, PAIR), jnp.float32),
        mesh=mesh,
        scratch_types=[
            pltpu.VMEM((2 * ROWS_PER_WORKER, HALF_PAD), jnp.int32),
            pltpu.VMEM((NBUF_G, 2, HALF_PAD, N_DIMS), jnp.float32),
            pltpu.VMEM((NBUF_O, HALF, 1, PAIR), jnp.float32),
            pltpu.VMEM((HALF, PAIR), jnp.float32),
            pltpu.SemaphoreType.DMA((NBUF_G,)),
            pltpu.SemaphoreType.DMA((NBUF_O,)),
        ],
        compiler_params=pltpu.CompilerParams(use_tc_tiling_on_sc=False),
    )
    out4 = f(resp_prep, pos_pair, response_embed)
    return out4.transpose(0, 2, 1, 3).reshape(BATCH, SEQ_LEN, N_DIMS)


def kernel(response, position_embed, response_embed):
    response = response.astype(jnp.int32)
    pad = jnp.zeros((BATCH, HALF_PAD - HALF), jnp.int32)
    resp_prep = jnp.concatenate(
        [response[:, 0::2], pad, response[:, 1::2], pad], axis=1
    ).reshape(2 * BATCH, HALF_PAD)
    pos_pair = position_embed.reshape(HALF, PAIR)
    return _run(resp_prep, pos_pair, response_embed)


# speedup vs baseline: 2.4232x; 2.4232x over previous
import jax
import jax.numpy as jnp
from jax import lax
from jax.experimental import pallas as pl
from jax.experimental.pallas import tpu as pltpu
from jax.experimental.pallas import tpu_sc as plsc

SEQ_LEN = 200
N_DIMS = 64
BATCH = 4096

NUM_CORES = 2
NUM_SUBCORES = 16
NUM_WORKERS = NUM_CORES * NUM_SUBCORES
ROWS_PER_WORKER = BATCH // NUM_WORKERS

NBUF = 4
LOOKAHEAD = 2


def _body(resp_t_hbm, pos_hbm, tab_hbm, out_hbm, idx_all, rows, pos_v,
          sem_g, sem_o):
    wid = lax.axis_index("s") * NUM_CORES + lax.axis_index("c")
    b0 = wid * ROWS_PER_WORKER

    pltpu.sync_copy(pos_hbm, pos_v)
    pltpu.sync_copy(resp_t_hbm.at[:, pl.ds(b0, ROWS_PER_WORKER)], idx_all)

    def out_slice(s):
        return out_hbm.at[pl.ds(b0, ROWS_PER_WORKER), s]

    def start_gather(s, k):
        pltpu.async_copy(tab_hbm.at[idx_all.at[s]], rows.at[k], sem_g.at[k])

    for s in range(LOOKAHEAD):
        start_gather(s, s)

    def step(i, carry):
        s0 = i * NBUF
        for k in range(NBUF):
            s = s0 + k
            cur = rows.at[k]
            pltpu.make_async_copy(
                tab_hbm.at[idx_all.at[s]], cur, sem_g.at[k]
            ).wait()

            s_pre = s + LOOKAHEAD - NBUF
            k_nxt = (k + LOOKAHEAD) % NBUF

            @pl.when(s_pre >= 0)
            def _():
                pltpu.make_async_copy(
                    rows.at[k_nxt], out_slice(s_pre), sem_o.at[k_nxt]
                ).wait()

            @pl.when(s + LOOKAHEAD <= SEQ_LEN - 1)
            def _():
                start_gather(s + LOOKAHEAD, k_nxt)

            p = [pos_v[s, pl.ds(16 * l, 16)] for l in range(N_DIMS // 16)]

            def add_row(r, c):
                for l in range(N_DIMS // 16):
                    sl = pl.ds(16 * l, 16)
                    cur[r, sl] = cur[r, sl] + p[l]
                return c

            lax.fori_loop(0, ROWS_PER_WORKER, add_row, 0, unroll=4)

            pltpu.async_copy(cur, out_slice(s), sem_o.at[k])
        return carry

    lax.fori_loop(0, SEQ_LEN // NBUF, step, 0)
    for s in range(SEQ_LEN - NBUF + LOOKAHEAD, SEQ_LEN):
        k = s % NBUF
        pltpu.make_async_copy(rows.at[k], out_slice(s), sem_o.at[k]).wait()


@jax.jit
def _run(resp_t, position_embed, response_embed):
    mesh = plsc.VectorSubcoreMesh(core_axis_name="c", subcore_axis_name="s")
    f = pl.kernel(
        _body,
        out_type=jax.ShapeDtypeStruct((BATCH, SEQ_LEN, N_DIMS), jnp.float32),
        mesh=mesh,
        scratch_types=[
            pltpu.VMEM((SEQ_LEN, ROWS_PER_WORKER), jnp.int32),
            pltpu.VMEM((NBUF, ROWS_PER_WORKER, N_DIMS), jnp.float32),
            pltpu.VMEM((SEQ_LEN, N_DIMS), jnp.float32),
            pltpu.SemaphoreType.DMA((NBUF,)),
            pltpu.SemaphoreType.DMA((NBUF,)),
        ],
        compiler_params=pltpu.CompilerParams(use_tc_tiling_on_sc=False),
    )
    return f(resp_t, position_embed, response_embed)


def kernel(response, position_embed, response_embed):
    resp_t = response.astype(jnp.int32).T
    return _run(resp_t, position_embed, response_embed)
